# baseline (speedup 1.0000x reference)
import jax
import jax.numpy as jnp
from jax import lax
from jax.experimental import pallas as pl
from jax.experimental.pallas import tpu as pltpu
from jax.experimental.pallas import tpu_sc as plsc

J = 24
L = 64
C = 16
N = 16384
HW = L * L
D = J * C
OUTC = 3 * D
NC, NS, LANES = 2, 16, 16
NW = NC * NS
PTS = N // NW
CH = LANES
NCHUNK = PTS // CH
MB = 4096


def _splat_i32(v):
    return jnp.full((LANES,), v, jnp.int32)


def _lane_perm(v, idx):
    return lax.gather(
        v, idx[:, None],
        dimension_numbers=lax.GatherDimensionNumbers(
            offset_dims=(), collapsed_slice_dims=(0,), start_index_map=(0,)),
        slice_sizes=(1,), mode=lax.GatherScatterMode.PROMISE_IN_BOUNDS)


def _lane_sum_splat(v):
    idx = lax.iota(jnp.int32, LANES)
    for sh in (1, 2, 4, 8):
        v = v + _lane_perm(v, idx ^ sh)
    return v


def _sc_body(qx_hbm, qy_hbm, qz_hbm, scale_hbm, tabx, taby, tabz, out_hbm,
             cx_v, cy_v, cz_v, scale_v, mbuf,
             idx0, idx1, idx2, wrefA, wrefB, wrefC,
             rows0, rows1, rows2, outstA, outstB,
             gs0, gs1, gs2, osA, osB):
    wid = lax.axis_index("s") * NC + lax.axis_index("c")
    base = wid * PTS
    tabs = (tabx, taby, tabz)
    idxbufs = (idx0, idx1, idx2)
    rowsbufs = (rows0, rows1, rows2)
    wrefs = (wrefA, wrefB, wrefC)
    gsems = (gs0, gs1, gs2)
    outsts = (outstA, outstB)
    osems = (osA, osB)

    means = []
    for q_hbm in (qx_hbm, qy_hbm, qz_hbm):
        acc = jnp.zeros((LANES,), jnp.float32)
        for b in range(N // MB):
            pltpu.sync_copy(q_hbm.at[pl.ds(b * MB, MB)], mbuf)

            def red(i, a):
                return a + mbuf[pl.ds(i * LANES, LANES)]

            acc = lax.fori_loop(0, MB // LANES, red, acc)
        means.append(_lane_sum_splat(acc) * (1.0 / N))
    mx, my, mz = means

    pltpu.sync_copy(scale_hbm, scale_v)
    half_sx = scale_v[pl.ds(0, LANES)] * 0.5
    half_sy = scale_v[pl.ds(LANES, LANES)] * 0.5
    half_sz = scale_v[pl.ds(2 * LANES, LANES)] * 0.5

    pltpu.sync_copy(qx_hbm.at[pl.ds(base, PTS)], cx_v)
    pltpu.sync_copy(qy_hbm.at[pl.ds(base, PTS)], cy_v)
    pltpu.sync_copy(qz_hbm.at[pl.ds(base, PTS)], cz_v)

    def build_fire(g, p, slot):
        pairs = ((cx_v, half_sx, mx), (cy_v, half_sy, my), (cz_v, half_sz, mz))
        ri, ci = ((0, 1), (1, 2), (2, 0))[p]

        def coord(i):
            cv, hs, m = pairs[i]
            return (cv[pl.ds(g * CH, CH)] - m) / hs

        rowc = coord(ri)
        colc = coord(ci)
        rf = jnp.clip((rowc + 1.0) * (0.5 * (L - 1)), 0.0, L - 1.0)
        cf = jnp.clip((colc + 1.0) * (0.5 * (L - 1)), 0.0, L - 1.0)
        r0 = rf.astype(jnp.int32)
        c0 = cf.astype(jnp.int32)
        rw = rf - r0.astype(jnp.float32)
        cb = jnp.minimum(c0, L - 2)
        fx = cf - cb.astype(jnp.float32)
        r1 = jnp.minimum(r0 + 1, L - 1)
        ib = idxbufs[slot]
        ib[pl.ds(0, CH)] = r0 * L + cb
        ib[pl.ds(CH, CH)] = r1 * L + cb
        wref = wrefs[slot]
        wref[0, :] = (1.0 - rw) * (1.0 - fx)
        wref[1, :] = (1.0 - rw) * fx
        wref[2, :] = rw * (1.0 - fx)
        wref[3, :] = rw * fx
        pltpu.async_copy(tabs[p].at[ib], rowsbufs[slot], gsems[slot])

    def compute_plane(p, slot, outst):
        off = p * C
        rows_v = rowsbufs[slot]
        wref = wrefs[slot]

        @plsc.parallel_loop(0, CH, 1, unroll=2)
        def _(pt):
            pv = jnp.full((LANES,), pt, jnp.int32)
            w00 = plsc.load_gather(wref, [_splat_i32(0), pv])
            w01 = plsc.load_gather(wref, [_splat_i32(1), pv])
            w10 = plsc.load_gather(wref, [_splat_i32(2), pv])
            w11 = plsc.load_gather(wref, [_splat_i32(3), pv])

            def texel(row, half, k):
                w = rows_v[row, pl.ds(half * D // 2 + k * C, C)]
                return plsc.unpack(plsc.bitcast(w, jnp.bfloat16),
                                   format=plsc.PackFormat.INTERLEAVED)

            for k in range(J // 2):
                a00, b00 = texel(pt, 0, k)
                a01, b01 = texel(pt, 1, k)
                a10, b10 = texel(pt + CH, 0, k)
                a11, b11 = texel(pt + CH, 1, k)
                oa = a00 * w00 + a01 * w01 + a10 * w10 + a11 * w11
                ob = b00 * w00 + b01 * w01 + b10 * w10 + b11 * w11
                outst[pt, pl.ds(96 * k + off, C)] = oa
                outst[pt, pl.ds(96 * k + 48 + off, C)] = ob

    build_fire(0, 0, 0)
    build_fire(0, 1, 1)

    def t_body(t, _):
        for k in range(6):
            p = k % 3
            g = 2 * t + k // 3
            slot = k % 3
            nslot = (k + 2) % 3
            nplane = (k + 2) % 3
            ng = 2 * t + (k + 2) // 3
            oslot = k // 3
            pltpu.make_async_copy(tabs[p].at[idxbufs[slot]], rowsbufs[slot],
                                  gsems[slot]).wait()
            if k < 4:
                build_fire(ng, nplane, nslot)
            else:
                @pl.when(ng < NCHUNK)
                def _(ng=ng, nplane=nplane, nslot=nslot):
                    build_fire(ng, nplane, nslot)
            if p == 0:
                @pl.when(t > 0)
                def _(oslot=oslot):
                    pltpu.make_async_copy(outsts[oslot],
                                          out_hbm.at[pl.ds(base, CH)],
                                          osems[oslot]).wait()
            compute_plane(p, slot, outsts[oslot])
            if p == 2:
                pltpu.async_copy(outsts[oslot],
                                 out_hbm.at[pl.ds(base + g * CH, CH)],
                                 osems[oslot])
        return 0

    lax.fori_loop(0, NCHUNK // 2, t_body, 0)
    pltpu.make_async_copy(outstA, out_hbm.at[pl.ds(base, CH)], osA).wait()
    pltpu.make_async_copy(outstB, out_hbm.at[pl.ds(base, CH)], osB).wait()


@jax.jit
def _run(qx, qy, qz, scale_pad, tabx, taby, tabz):
    mesh = plsc.VectorSubcoreMesh(core_axis_name="c", subcore_axis_name="s",
                                  num_cores=NC, num_subcores=NS)
    f = pl.kernel(
        _sc_body,
        out_type=jax.ShapeDtypeStruct((N, OUTC), jnp.float32),
        mesh=mesh,
        compiler_params=pltpu.CompilerParams(needs_layout_passes=False),
        scratch_types=[
            pltpu.VMEM((PTS,), jnp.float32),
            pltpu.VMEM((PTS,), jnp.float32),
            pltpu.VMEM((PTS,), jnp.float32),
            pltpu.VMEM((3 * LANES,), jnp.float32),
            pltpu.VMEM((MB,), jnp.float32),
            pltpu.VMEM((2 * CH,), jnp.int32),
            pltpu.VMEM((2 * CH,), jnp.int32),
            pltpu.VMEM((2 * CH,), jnp.int32),
            pltpu.VMEM((4, LANES), jnp.float32),
            pltpu.VMEM((4, LANES), jnp.float32),
            pltpu.VMEM((4, LANES), jnp.float32),
            pltpu.VMEM((2 * CH, D), jnp.int32),
            pltpu.VMEM((2 * CH, D), jnp.int32),
            pltpu.VMEM((2 * CH, D), jnp.int32),
            pltpu.VMEM((CH, OUTC), jnp.float32),
            pltpu.VMEM((CH, OUTC), jnp.float32),
            pltpu.SemaphoreType.DMA,
            pltpu.SemaphoreType.DMA,
            pltpu.SemaphoreType.DMA,
            pltpu.SemaphoreType.DMA,
            pltpu.SemaphoreType.DMA,
        ],
    )
    return f(qx, qy, qz, scale_pad, tabx, taby, tabz)


def kernel(id, query_points, scale, feat_lines_x, feat_lines_y, feat_lines_z):
    qx = query_points[:, 0]
    qy = query_points[:, 1]
    qz = query_points[:, 2]
    scale_pad = jnp.repeat(scale.astype(jnp.float32), LANES)

    def prep(t):
        b = jnp.transpose(t[:, id], (1, 2, 0, 3)).reshape(HW, J // 2, 2, C)
        b = jnp.transpose(b, (0, 1, 3, 2)).reshape(HW, D)
        b = b.astype(jnp.bfloat16)
        nxt = jnp.concatenate([b[1:], jnp.zeros((1, D), jnp.bfloat16)], 0)
        pair = jnp.concatenate([b, nxt], axis=1)
        return jax.lax.bitcast_convert_type(
            pair.reshape(HW, D, 2), jnp.int32)

    out = _run(qx, qy, qz, scale_pad, prep(feat_lines_x), prep(feat_lines_y),
               prep(feat_lines_z))
    return out.reshape(1, N, OUTC)


# speedup vs baseline: 1.1305x; 1.1305x over previous
import jax
import jax.numpy as jnp
from jax import lax
from jax.experimental import pallas as pl
from jax.experimental.pallas import tpu as pltpu
from jax.experimental.pallas import tpu_sc as plsc

J = 24
L = 64
C = 16
N = 16384
HW = L * L
D = J * C
OUTC = 3 * D
NC, NS, LANES = 2, 16, 16
NW = NC * NS
PTS = N // NW
CH = LANES
NCHUNK = PTS // CH
MB = 4096


def _splat_i32(v):
    return jnp.full((LANES,), v, jnp.int32)


def _lane_perm(v, idx):
    return lax.gather(
        v, idx[:, None],
        dimension_numbers=lax.GatherDimensionNumbers(
            offset_dims=(), collapsed_slice_dims=(0,), start_index_map=(0,)),
        slice_sizes=(1,), mode=lax.GatherScatterMode.PROMISE_IN_BOUNDS)


def _lane_sum_splat(v):
    idx = lax.iota(jnp.int32, LANES)
    for sh in (1, 2, 4, 8):
        v = v + _lane_perm(v, idx ^ sh)
    return v


def _sc_body(qx_hbm, qy_hbm, qz_hbm, scale_hbm, tabx, taby, tabz, out_hbm,
             cx_v, cy_v, cz_v, scale_v, mbuf,
             idx0, idx1, idx2, wrefA, wrefB, wrefC,
             rows0, rows1, rows2, outstA, outstB,
             gs0, gs1, gs2, osA, osB):
    wid = lax.axis_index("s") * NC + lax.axis_index("c")
    base = wid * PTS
    tabs = (tabx, taby, tabz)
    idxbufs = (idx0, idx1, idx2)
    rowsbufs = (rows0, rows1, rows2)
    wrefs = (wrefA, wrefB, wrefC)
    gsems = (gs0, gs1, gs2)
    outsts = (outstA, outstB)
    osems = (osA, osB)

    means = []
    for q_hbm in (qx_hbm, qy_hbm, qz_hbm):
        acc = jnp.zeros((LANES,), jnp.float32)
        for b in range(N // MB):
            pltpu.sync_copy(q_hbm.at[pl.ds(b * MB, MB)], mbuf)

            def red(i, a):
                return a + mbuf[pl.ds(i * LANES, LANES)]

            acc = lax.fori_loop(0, MB // LANES, red, acc)
        means.append(_lane_sum_splat(acc) * (1.0 / N))
    mx, my, mz = means

    pltpu.sync_copy(scale_hbm, scale_v)
    half_sx = scale_v[pl.ds(0, LANES)] * 0.5
    half_sy = scale_v[pl.ds(LANES, LANES)] * 0.5
    half_sz = scale_v[pl.ds(2 * LANES, LANES)] * 0.5

    pltpu.sync_copy(qx_hbm.at[pl.ds(base, PTS)], cx_v)
    pltpu.sync_copy(qy_hbm.at[pl.ds(base, PTS)], cy_v)
    pltpu.sync_copy(qz_hbm.at[pl.ds(base, PTS)], cz_v)

    def build_fire(g, p, slot):
        pairs = ((cx_v, half_sx, mx), (cy_v, half_sy, my), (cz_v, half_sz, mz))
        ri, ci = ((0, 1), (1, 2), (2, 0))[p]

        def coord(i):
            cv, hs, m = pairs[i]
            return (cv[pl.ds(g * CH, CH)] - m) / hs

        rowc = coord(ri)
        colc = coord(ci)
        rf = jnp.clip((rowc + 1.0) * (0.5 * (L - 1)), 0.0, L - 1.0)
        cf = jnp.clip((colc + 1.0) * (0.5 * (L - 1)), 0.0, L - 1.0)
        r0 = rf.astype(jnp.int32)
        c0 = cf.astype(jnp.int32)
        rw = rf - r0.astype(jnp.float32)
        cb = jnp.minimum(c0, L - 2)
        fx = cf - cb.astype(jnp.float32)
        r1 = jnp.minimum(r0 + 1, L - 1)
        ib = idxbufs[slot]
        ib[pl.ds(0, CH)] = r0 * L + cb
        ib[pl.ds(CH, CH)] = r1 * L + cb
        wref = wrefs[slot]
        wref[0, :] = (1.0 - rw) * (1.0 - fx)
        wref[1, :] = (1.0 - rw) * fx
        wref[2, :] = rw * (1.0 - fx)
        wref[3, :] = rw * fx
        pltpu.async_copy(tabs[p].at[ib], rowsbufs[slot], gsems[slot])

    def compute_plane(p, slot, outst):
        off = p * C
        rows_v = rowsbufs[slot]
        wref = wrefs[slot]

        @plsc.parallel_loop(0, CH, 1, unroll=2)
        def _(pt):
            pv = jnp.full((LANES,), pt, jnp.int32)
            w00 = plsc.load_gather(wref, [_splat_i32(0), pv])
            w01 = plsc.load_gather(wref, [_splat_i32(1), pv])
            w10 = plsc.load_gather(wref, [_splat_i32(2), pv])
            w11 = plsc.load_gather(wref, [_splat_i32(3), pv])

            def texel(row, half, k):
                w = rows_v[row, pl.ds(half * D // 2 + k * C, C)]
                return plsc.unpack(plsc.bitcast(w, jnp.bfloat16),
                                   format=plsc.PackFormat.INTERLEAVED)

            for k in range(J // 2):
                a00, b00 = texel(pt, 0, k)
                a01, b01 = texel(pt, 1, k)
                a10, b10 = texel(pt + CH, 0, k)
                a11, b11 = texel(pt + CH, 1, k)
                oa = a00 * w00 + a01 * w01 + a10 * w10 + a11 * w11
                ob = b00 * w00 + b01 * w01 + b10 * w10 + b11 * w11
                outst[pt, pl.ds(96 * k + off, C)] = oa
                outst[pt, pl.ds(96 * k + 48 + off, C)] = ob

    build_fire(0, 0, 0)
    build_fire(0, 1, 1)

    def t_body(t, _):
        for k in range(6):
            p = k % 3
            g = 2 * t + k // 3
            slot = k % 3
            nslot = (k + 2) % 3
            nplane = (k + 2) % 3
            ng = 2 * t + (k + 2) // 3
            oslot = k // 3
            pltpu.make_async_copy(tabs[p].at[idxbufs[slot]], rowsbufs[slot],
                                  gsems[slot]).wait()
            if k < 4:
                build_fire(ng, nplane, nslot)
            else:
                @pl.when(ng < NCHUNK)
                def _(ng=ng, nplane=nplane, nslot=nslot):
                    build_fire(ng, nplane, nslot)
            if p == 0:
                @pl.when(t > 0)
                def _(oslot=oslot):
                    pltpu.make_async_copy(outsts[oslot],
                                          out_hbm.at[pl.ds(base, CH)],
                                          osems[oslot]).wait()
            compute_plane(p, slot, outsts[oslot])
            if p == 2:
                pltpu.async_copy(outsts[oslot],
                                 out_hbm.at[pl.ds(base + g * CH, CH)],
                                 osems[oslot])
        return 0

    lax.fori_loop(0, NCHUNK // 2, t_body, 0)
    pltpu.make_async_copy(outstA, out_hbm.at[pl.ds(base, CH)], osA).wait()
    pltpu.make_async_copy(outstB, out_hbm.at[pl.ds(base, CH)], osB).wait()


@jax.jit
def _run(qx, qy, qz, scale_pad, tabx, taby, tabz):
    mesh = plsc.VectorSubcoreMesh(core_axis_name="c", subcore_axis_name="s",
                                  num_cores=NC, num_subcores=NS)
    f = pl.kernel(
        _sc_body,
        out_type=jax.ShapeDtypeStruct((N, OUTC), jnp.float32),
        mesh=mesh,
        compiler_params=pltpu.CompilerParams(needs_layout_passes=False),
        scratch_types=[
            pltpu.VMEM((PTS,), jnp.float32),
            pltpu.VMEM((PTS,), jnp.float32),
            pltpu.VMEM((PTS,), jnp.float32),
            pltpu.VMEM((3 * LANES,), jnp.float32),
            pltpu.VMEM((MB,), jnp.float32),
            pltpu.VMEM((2 * CH,), jnp.int32),
            pltpu.VMEM((2 * CH,), jnp.int32),
            pltpu.VMEM((2 * CH,), jnp.int32),
            pltpu.VMEM((4, LANES), jnp.float32),
            pltpu.VMEM((4, LANES), jnp.float32),
            pltpu.VMEM((4, LANES), jnp.float32),
            pltpu.VMEM((2 * CH, D), jnp.int32),
            pltpu.VMEM((2 * CH, D), jnp.int32),
            pltpu.VMEM((2 * CH, D), jnp.int32),
            pltpu.VMEM((CH, OUTC), jnp.float32),
            pltpu.VMEM((CH, OUTC), jnp.float32),
            pltpu.SemaphoreType.DMA,
            pltpu.SemaphoreType.DMA,
            pltpu.SemaphoreType.DMA,
            pltpu.SemaphoreType.DMA,
            pltpu.SemaphoreType.DMA,
        ],
    )
    return f(qx, qy, qz, scale_pad, tabx, taby, tabz)


def kernel(id, query_points, scale, feat_lines_x, feat_lines_y, feat_lines_z):
    qx = query_points[:, 0]
    qy = query_points[:, 1]
    qz = query_points[:, 2]
    scale_pad = jnp.repeat(scale.astype(jnp.float32), LANES)

    def prep(t):
        b = t[:, id].astype(jnp.bfloat16).reshape(J // 2, 2, L, L, C)
        b = jnp.transpose(b, (2, 3, 0, 4, 1)).reshape(HW, D)
        nxt = jnp.concatenate([b[1:], jnp.zeros((1, D), jnp.bfloat16)], 0)
        pair = jnp.concatenate([b, nxt], axis=1)
        return jax.lax.bitcast_convert_type(
            pair.reshape(HW, D, 2), jnp.int32)

    out = _run(qx, qy, qz, scale_pad, prep(feat_lines_x), prep(feat_lines_y),
               prep(feat_lines_z))
    return out.reshape(1, N, OUTC)
